# Initial kernel scaffold; baseline (speedup 1.0000x reference)
import jax
import jax.numpy as jnp
from jax.experimental import pallas as pl


def kernel(xy, params):
    raise NotImplementedError("write your pallas kernel here")


# speedup vs baseline: 3.0977x; 3.0977x over previous
import functools
import math

import jax
import jax.numpy as jnp
from jax import lax
from jax.experimental import pallas as pl
from jax.experimental.pallas import tpu as pltpu
from jax.experimental.pallas import tpu_sc as plsc

_N_LEVELS = 16
_N_FEATURES = 2
_LOG2_HASHMAP = 19
_BASE_RES = 16
_PER_LEVEL_SCALE = 1.5


def _aligned_size(resolution):
    p = resolution * resolution
    p = (p + 7) // 8 * 8
    return min(p, 1 << _LOG2_HASHMAP)


def _levels():
    res, offs = [], []
    offset = 0
    log2_scale = math.log2(_PER_LEVEL_SCALE)
    for level in range(_N_LEVELS):
        scale = 2.0 ** (level * log2_scale) * _BASE_RES - 1.0
        r = int(math.ceil(scale) + 1)
        res.append(r)
        offs.append(offset)
        offset += _aligned_size(r)
    return res, offs, offset


_RES, _OFF, _TOTAL = _levels()
_HSIZE = [
    (_OFF[l + 1] if l + 1 < _N_LEVELS else _TOTAL) - _OFF[l]
    for l in range(_N_LEVELS)
]
_IS_DENSE = [_HSIZE[l] >= _RES[l] * _RES[l] for l in range(_N_LEVELS)]
_PRIME_I32 = -1640531535

_NC, _NS, _L = 2, 16, 16
_NW = _NC * _NS


def _build_sc_kernel(n_points, chunk):
    ppw = n_points // _NW
    nchunk = ppw // chunk
    nvec = chunk // _L

    mesh = plsc.VectorSubcoreMesh(core_axis_name="c", subcore_axis_name="s")

    @functools.partial(
        pl.kernel,
        mesh=mesh,
        out_type=jax.ShapeDtypeStruct((n_points, 2 * _N_LEVELS), jnp.float32),
        scratch_types=[
            pltpu.VMEM((chunk,), jnp.float32),
            pltpu.VMEM((chunk,), jnp.float32),
            pltpu.VMEM((chunk, 2 * _N_LEVELS), jnp.float32),
            [pltpu.VMEM((chunk,), jnp.int32) for _ in range(4)],
            [pltpu.VMEM((chunk, 2), jnp.float32) for _ in range(4)],
            pltpu.SemaphoreType.DMA,
        ],
        compiler_params=pltpu.CompilerParams(use_tc_tiling_on_sc=False, needs_layout_passes=False),
    )
    def grid_kernel(xt_hbm, table_hbm, out_hbm, x_v, y_v, out_v, idx_vs,
                    rows_vs, sem):
        i32 = lambda v: jnp.int32(v)
        wid = (lax.axis_index("s") * i32(_NC) + lax.axis_index("c")).astype(
            jnp.int32)
        base0 = wid * i32(ppw)
        iota = lax.iota(jnp.int32, _L)
        col0 = jnp.zeros((_L,), jnp.int32)
        col1 = jnp.ones((_L,), jnp.int32)

        @pl.loop(jnp.int32(0), jnp.int32(nchunk))
        def _chunk(ci):
            base = base0 + ci.astype(jnp.int32) * i32(chunk)
            pltpu.sync_copy(xt_hbm.at[i32(0), pl.ds(base, chunk)], x_v)
            pltpu.sync_copy(xt_hbm.at[i32(1), pl.ds(base, chunk)], y_v)

            for lv in range(_N_LEVELS):
                res = _RES[lv]
                off = _OFF[lv]
                scale = float(res - 1.0)

                @pl.loop(jnp.int32(0), jnp.int32(nvec))
                def _pass_a(i, lv=lv, res=res, off=off, scale=scale):
                    s = pl.ds(i.astype(jnp.int32) * i32(_L), _L)
                    px = x_v[s] * scale + 0.5
                    py = y_v[s] * scale + 0.5
                    gx = px.astype(jnp.int32)
                    gy = py.astype(jnp.int32)
                    if _IS_DENSE[lv]:
                        b = off + gx + gy * res
                        idx_vs[0][s] = b
                        idx_vs[1][s] = b + 1
                        idx_vs[2][s] = b + res
                        idx_vs[3][s] = b + (res + 1)
                    else:
                        mask = _HSIZE[lv] - 1
                        t0 = gy * _PRIME_I32
                        t1 = t0 + _PRIME_I32
                        gx1 = gx + 1
                        idx_vs[0][s] = off + ((gx ^ t0) & mask)
                        idx_vs[1][s] = off + ((gx1 ^ t0) & mask)
                        idx_vs[2][s] = off + ((gx ^ t1) & mask)
                        idx_vs[3][s] = off + ((gx1 ^ t1) & mask)

                cps = [
                    pltpu.async_copy(table_hbm.at[idx_vs[k]], rows_vs[k], sem)
                    for k in range(4)
                ]
                for cp in cps:
                    cp.wait()

                @pl.loop(jnp.int32(0), jnp.int32(nvec))
                def _pass_b(i, lv=lv, scale=scale):
                    o = i.astype(jnp.int32) * i32(_L)
                    s = pl.ds(o, _L)
                    px = x_v[s] * scale + 0.5
                    py = y_v[s] * scale + 0.5
                    wx = px - px.astype(jnp.int32).astype(jnp.float32)
                    wy = py - py.astype(jnp.int32).astype(jnp.float32)
                    omx = 1.0 - wx
                    omy = 1.0 - wy
                    w = (omx * omy, wx * omy, omx * wy, wx * wy)
                    ridx = o + iota
                    acc0 = plsc.load_gather(rows_vs[0], [ridx, col0]) * w[0]
                    acc1 = plsc.load_gather(rows_vs[0], [ridx, col1]) * w[0]
                    for k in range(1, 4):
                        acc0 += plsc.load_gather(rows_vs[k], [ridx, col0]) * w[k]
                        acc1 += plsc.load_gather(rows_vs[k], [ridx, col1]) * w[k]
                    plsc.store_scatter(out_v, [ridx, col0 + (2 * lv)], acc0)
                    plsc.store_scatter(out_v, [ridx, col0 + (2 * lv + 1)], acc1)

            pltpu.sync_copy(out_v, out_hbm.at[pl.ds(base, chunk)])

    return grid_kernel


@jax.jit
def kernel(xy, params):
    n = xy.shape[0]
    xt = xy.astype(jnp.float32).T
    table = params.astype(jnp.float32).reshape(_TOTAL, _N_FEATURES)
    out = _build_sc_kernel(n, 1024)(xt, table)
    return out.astype(xy.dtype)
